# baseline (speedup 1.0000x reference)
import functools
import jax
import jax.numpy as jnp
from jax import lax
from jax.experimental import pallas as pl
from jax.experimental.pallas import tpu as pltpu
from jax.experimental.pallas import tpu_sc as plsc

N_BINS = 15
NSTREAM = 4
BLOCK_ROWS = 2000

N_TOTAL = 1000000
SC_ROWS = 384000
NTC = N_TOTAL - SC_ROWS
SC_WORKERS = 32
SC_RPW = SC_ROWS // SC_WORKERS
SC_CH = 480


def _recip_bounds():
    lane = lax.broadcasted_iota(jnp.int32, (1, 128), 1)
    lane_f = lane.astype(jnp.float32)
    b = jnp.where(lane == 0, jnp.float32(3.0e38), 15.0 / lane_f)
    return jnp.where(lane <= N_BINS, b, jnp.float32(-1.0))


def _tc_kernel(l0, l1, l2, l3, labT_ref, out_ref, acc_ref, *, nsteps):
    i = pl.program_id(0)

    @pl.when(i == 0)
    def _init():
        acc_ref[...] = jnp.zeros_like(acc_ref)

    bounds = _recip_bounds()
    nblocks = NSTREAM * nsteps
    row = lax.broadcasted_iota(jnp.int32, (nblocks, NSTREAM), 0)
    col = lax.broadcasted_iota(jnp.int32, (nblocks, NSTREAM), 1)
    sel = (row == NSTREAM * i + col).astype(jnp.float32)
    lab4 = lax.dot_general(labT_ref[...], sel, (((1,), (0,)), ((), ())),
                           preferred_element_type=jnp.float32)

    for k, ref in enumerate((l0, l1, l2, l3)):
        x = ref[...]
        r = x.shape[0]
        m = jnp.max(x, axis=1, keepdims=True)
        e = jnp.exp(x - m)
        s = jnp.sum(e, axis=1, keepdims=True)
        conf = 1.0 / s
        pred = jnp.argmax(x, axis=1).reshape(r, 1)
        lab = lab4[:, k:k + 1]
        acc = (pred.astype(jnp.float32) == lab).astype(jnp.float32)

        gt = (s < bounds).astype(jnp.float32)
        acc_ref[0:1, :] += jnp.sum(gt, axis=0, keepdims=True)
        acc_ref[1:2, :] += jnp.sum(gt * acc, axis=0, keepdims=True)
        acc_ref[2:3, :] += jnp.sum(gt * conf, axis=0, keepdims=True)

    @pl.when(i == nsteps - 1)
    def _finish():
        out_ref[...] = acc_ref[...]


def _tc_partial(logits, labels):
    nblocks = NTC // BLOCK_ROWS
    nsteps = nblocks // NSTREAM
    c = logits.shape[1]
    labT = labels[:NTC].astype(jnp.float32).reshape(nblocks, BLOCK_ROWS).T

    return pl.pallas_call(
        functools.partial(_tc_kernel, nsteps=nsteps),
        grid=(nsteps,),
        in_specs=[
            pl.BlockSpec((BLOCK_ROWS, c), lambda i: (NSTREAM * i, 0)),
            pl.BlockSpec((BLOCK_ROWS, c), lambda i: (NSTREAM * i + 1, 0)),
            pl.BlockSpec((BLOCK_ROWS, c), lambda i: (NSTREAM * i + 2, 0)),
            pl.BlockSpec((BLOCK_ROWS, c), lambda i: (NSTREAM * i + 3, 0)),
            pl.BlockSpec((BLOCK_ROWS, nblocks), lambda i: (0, 0)),
        ],
        out_specs=pl.BlockSpec((8, 128), lambda i: (0, 0)),
        out_shape=jax.ShapeDtypeStruct((8, 128), jnp.float32),
        scratch_shapes=[pltpu.VMEM((8, 128), jnp.float32)],
    )(logits, logits, logits, logits, labT)


def _sc_body(logits_hbm, labels_hbm, out_hbm, xbuf, lbuf, bins):
    cc = lax.axis_index("c")
    ss = lax.axis_index("s")
    wid = ss * 2 + cc
    base = NTC + wid * SC_RPW

    zero16 = jnp.zeros((16,), jnp.float32)
    for j in range(3):
        bins[pl.ds(j * 16, 16)] = zero16

    lane = lax.broadcasted_iota(jnp.int32, (16,), 0)
    ones16 = jnp.ones((16,), jnp.float32)
    neg_inf = jnp.full((16,), -3.0e38, jnp.float32)

    def chunk(t, _):
        r0 = base + t * SC_CH
        pltpu.sync_copy(logits_hbm.at[pl.ds(r0, SC_CH)], xbuf)
        pltpu.sync_copy(labels_hbm.at[pl.ds(r0, SC_CH)], lbuf)

        def group(g, __):
            rows16 = g * 16 + lane

            def colstep(cidx, carry):
                mval, prd, sacc = carry
                v = plsc.load_gather(xbuf, [rows16, jnp.zeros((16,), jnp.int32) + cidx])
                bigger = v > mval
                prd = jnp.where(bigger, jnp.full((16,), 1, jnp.int32) * cidx, prd)
                mval = jnp.maximum(mval, v)
                sacc = sacc + jnp.exp(v)
                return mval, prd, sacc

            mval, prd, sacc = lax.fori_loop(
                0, 100, colstep,
                (neg_inf, jnp.zeros((16,), jnp.int32), zero16))

            conf = jnp.exp(mval) / sacc
            lab = lbuf[pl.ds(g * 16, 16)]
            accv = jnp.where(prd == lab, 1.0, 0.0).astype(jnp.float32)
            t15 = conf * 15.0
            ti = t15.astype(jnp.int32)
            tf = ti.astype(jnp.float32)
            b = jnp.where(tf == t15, ti - 1, ti)
            plsc.addupdate_scatter(bins, [b], ones16)
            plsc.addupdate_scatter(bins, [b + 16], accv)
            plsc.addupdate_scatter(bins, [b + 32], conf)
            return __

        lax.fori_loop(0, SC_CH // 16, group, 0)
        return _

    lax.fori_loop(0, SC_RPW // SC_CH, chunk, 0)
    pltpu.sync_copy(bins, out_hbm.at[wid])


def _sc_partial(logits, labels):
    mesh = plsc.VectorSubcoreMesh(core_axis_name="c", subcore_axis_name="s")
    return pl.kernel(
        _sc_body,
        mesh=mesh,
        compiler_params=pltpu.CompilerParams(needs_layout_passes=False),
        out_type=jax.ShapeDtypeStruct((SC_WORKERS, 48), jnp.float32),
        scratch_types=[
            pltpu.VMEM((SC_CH, 100), jnp.float32),
            pltpu.VMEM((SC_CH,), jnp.int32),
            pltpu.VMEM((48,), jnp.float32),
        ],
    )(logits, labels)


def _combine_kernel(tc_ref, sc_ref, out_ref):
    cum = tc_ref[...]
    cnt = cum[0:1, 0:N_BINS] - cum[0:1, 1:N_BINS + 1]
    asum = cum[1:2, 0:N_BINS] - cum[1:2, 1:N_BINS + 1]
    csum = cum[2:3, 0:N_BINS] - cum[2:3, 1:N_BINS + 1]

    scs = jnp.sum(sc_ref[...], axis=0, keepdims=True)
    cnt = cnt + scs[0:1, 0:N_BINS]
    asum = asum + scs[0:1, 16:16 + N_BINS]
    csum = csum + scs[0:1, 32:32 + N_BINS]

    prop = cnt / jnp.float32(N_TOTAL)
    safe = jnp.maximum(cnt, 1.0)
    per_bin = jnp.where(cnt > 0.0, jnp.abs(csum / safe - asum / safe) * prop, 0.0)
    out_ref[...] = jnp.sum(per_bin).reshape(1, 1)


def _combine(tc_cum, sc_part):
    return pl.pallas_call(
        _combine_kernel,
        out_shape=jax.ShapeDtypeStruct((1, 1), jnp.float32),
    )(tc_cum, sc_part)


def kernel(logits, labels):
    tc_cum = _tc_partial(logits, labels)
    sc_part = _sc_partial(logits, labels)
    return _combine(tc_cum, sc_part).reshape(1)


# speedup vs baseline: 1.3451x; 1.3451x over previous
import functools
import jax
import jax.numpy as jnp
from jax import lax
from jax.experimental import pallas as pl
from jax.experimental.pallas import tpu as pltpu
from jax.experimental.pallas import tpu_sc as plsc

N_BINS = 15
NSTREAM = 4
BLOCK_ROWS = 2000

N_TOTAL = 1000000
SC_ROWS = 384000
NTC = N_TOTAL - SC_ROWS
SC_WORKERS = 32
SC_RPW = SC_ROWS // SC_WORKERS
SC_CH = 480


def _recip_bounds():
    lane = lax.broadcasted_iota(jnp.int32, (1, 128), 1)
    lane_f = lane.astype(jnp.float32)
    b = jnp.where(lane == 0, jnp.float32(3.0e38), 15.0 / lane_f)
    return jnp.where(lane <= N_BINS, b, jnp.float32(-1.0))


def _tc_kernel(l0, l1, l2, l3, labT_ref, out_ref, acc_ref, *, nsteps):
    i = pl.program_id(0)

    @pl.when(i == 0)
    def _init():
        acc_ref[...] = jnp.zeros_like(acc_ref)

    bounds = _recip_bounds()
    nblocks = NSTREAM * nsteps
    row = lax.broadcasted_iota(jnp.int32, (nblocks, NSTREAM), 0)
    col = lax.broadcasted_iota(jnp.int32, (nblocks, NSTREAM), 1)
    sel = (row == NSTREAM * i + col).astype(jnp.float32)
    lab4 = lax.dot_general(labT_ref[...], sel, (((1,), (0,)), ((), ())),
                           preferred_element_type=jnp.float32)

    for k, ref in enumerate((l0, l1, l2, l3)):
        x = ref[...]
        r = x.shape[0]
        m = jnp.max(x, axis=1, keepdims=True)
        e = jnp.exp(x - m)
        s = jnp.sum(e, axis=1, keepdims=True)
        conf = 1.0 / s
        pred = jnp.argmax(x, axis=1).reshape(r, 1)
        lab = lab4[:, k:k + 1]
        acc = (pred.astype(jnp.float32) == lab).astype(jnp.float32)

        gt = (s < bounds).astype(jnp.float32)
        acc_ref[0:1, :] += jnp.sum(gt, axis=0, keepdims=True)
        acc_ref[1:2, :] += jnp.sum(gt * acc, axis=0, keepdims=True)
        acc_ref[2:3, :] += jnp.sum(gt * conf, axis=0, keepdims=True)

    @pl.when(i == nsteps - 1)
    def _finish():
        out_ref[...] = acc_ref[...]


def _tc_partial(logits, labels):
    nblocks = NTC // BLOCK_ROWS
    nsteps = nblocks // NSTREAM
    c = logits.shape[1]
    labT = labels[:NTC].astype(jnp.float32).reshape(nblocks, BLOCK_ROWS).T

    return pl.pallas_call(
        functools.partial(_tc_kernel, nsteps=nsteps),
        grid=(nsteps,),
        in_specs=[
            pl.BlockSpec((BLOCK_ROWS, c), lambda i: (NSTREAM * i, 0)),
            pl.BlockSpec((BLOCK_ROWS, c), lambda i: (NSTREAM * i + 1, 0)),
            pl.BlockSpec((BLOCK_ROWS, c), lambda i: (NSTREAM * i + 2, 0)),
            pl.BlockSpec((BLOCK_ROWS, c), lambda i: (NSTREAM * i + 3, 0)),
            pl.BlockSpec((BLOCK_ROWS, nblocks), lambda i: (0, 0)),
        ],
        out_specs=pl.BlockSpec((8, 128), lambda i: (0, 0)),
        out_shape=jax.ShapeDtypeStruct((8, 128), jnp.float32),
        scratch_shapes=[pltpu.VMEM((8, 128), jnp.float32)],
    )(logits, logits, logits, logits, labT)


def _sc_body(logits_hbm, labels_hbm, out_hbm, xbuf, lbuf, bins):
    cc = lax.axis_index("c")
    ss = lax.axis_index("s")
    wid = ss * 2 + cc
    base = NTC + wid * SC_RPW

    zero16 = jnp.zeros((16,), jnp.float32)
    for j in range(3):
        bins[pl.ds(j * 16, 16)] = zero16

    lane = lax.broadcasted_iota(jnp.int32, (16,), 0)
    ones16 = jnp.ones((16,), jnp.float32)
    neg_inf = jnp.full((16,), -3.0e38, jnp.float32)

    def chunk(t, _):
        r0 = base + t * SC_CH
        pltpu.sync_copy(logits_hbm.at[pl.ds(r0, SC_CH)], xbuf)
        pltpu.sync_copy(labels_hbm.at[pl.ds(r0, SC_CH)], lbuf)

        def group(g, __):
            rows16 = g * 16 + lane

            mval = neg_inf
            prd = jnp.zeros((16,), jnp.int32)
            sacc = zero16
            for cidx in range(100):
                v = plsc.load_gather(xbuf, [rows16, jnp.full((16,), cidx, jnp.int32)])
                bigger = v > mval
                prd = jnp.where(bigger, jnp.full((16,), cidx, jnp.int32), prd)
                mval = jnp.maximum(mval, v)
                sacc = sacc + jnp.exp(v)

            conf = jnp.exp(mval) / sacc
            lab = lbuf[pl.ds(g * 16, 16)]
            accv = jnp.where(prd == lab, 1.0, 0.0).astype(jnp.float32)
            t15 = conf * 15.0
            ti = t15.astype(jnp.int32)
            tf = ti.astype(jnp.float32)
            b = jnp.where(tf == t15, ti - 1, ti)
            plsc.addupdate_scatter(bins, [b], ones16)
            plsc.addupdate_scatter(bins, [b + 16], accv)
            plsc.addupdate_scatter(bins, [b + 32], conf)
            return __

        lax.fori_loop(0, SC_CH // 16, group, 0)
        return _

    lax.fori_loop(0, SC_RPW // SC_CH, chunk, 0)
    pltpu.sync_copy(bins, out_hbm.at[wid])


def _sc_partial(logits, labels):
    mesh = plsc.VectorSubcoreMesh(core_axis_name="c", subcore_axis_name="s")
    return pl.kernel(
        _sc_body,
        mesh=mesh,
        compiler_params=pltpu.CompilerParams(needs_layout_passes=False),
        out_type=jax.ShapeDtypeStruct((SC_WORKERS, 48), jnp.float32),
        scratch_types=[
            pltpu.VMEM((SC_CH, 100), jnp.float32),
            pltpu.VMEM((SC_CH,), jnp.int32),
            pltpu.VMEM((48,), jnp.float32),
        ],
    )(logits, labels)


def _combine_kernel(tc_ref, sc_ref, out_ref):
    cum = tc_ref[...]
    cnt = cum[0:1, 0:N_BINS] - cum[0:1, 1:N_BINS + 1]
    asum = cum[1:2, 0:N_BINS] - cum[1:2, 1:N_BINS + 1]
    csum = cum[2:3, 0:N_BINS] - cum[2:3, 1:N_BINS + 1]

    scs = jnp.sum(sc_ref[...], axis=0, keepdims=True)
    cnt = cnt + scs[0:1, 0:N_BINS]
    asum = asum + scs[0:1, 16:16 + N_BINS]
    csum = csum + scs[0:1, 32:32 + N_BINS]

    prop = cnt / jnp.float32(N_TOTAL)
    safe = jnp.maximum(cnt, 1.0)
    per_bin = jnp.where(cnt > 0.0, jnp.abs(csum / safe - asum / safe) * prop, 0.0)
    out_ref[...] = jnp.sum(per_bin).reshape(1, 1)


def _combine(tc_cum, sc_part):
    return pl.pallas_call(
        _combine_kernel,
        out_shape=jax.ShapeDtypeStruct((1, 1), jnp.float32),
    )(tc_cum, sc_part)


def kernel(logits, labels):
    sc_part = _sc_partial(logits, labels)
    tc_cum = _tc_partial(logits, labels)
    return _combine(tc_cum, sc_part).reshape(1)


# speedup vs baseline: 1.4331x; 1.0655x over previous
import functools
import jax
import jax.numpy as jnp
from jax import lax
from jax.experimental import pallas as pl
from jax.experimental.pallas import tpu as pltpu
from jax.experimental.pallas import tpu_sc as plsc

N_BINS = 15
NSTREAM = 4
BLOCK_ROWS = 2000

N_TOTAL = 1000000
SC_ROWS = 384000
NTC = N_TOTAL - SC_ROWS
SC_WORKERS = 32
SC_RPW = SC_ROWS // SC_WORKERS
SC_CH = 400


def _recip_bounds():
    lane = lax.broadcasted_iota(jnp.int32, (1, 128), 1)
    lane_f = lane.astype(jnp.float32)
    b = jnp.where(lane == 0, jnp.float32(3.0e38), 15.0 / lane_f)
    return jnp.where(lane <= N_BINS, b, jnp.float32(-1.0))


def _tc_kernel(l0, l1, l2, l3, labT_ref, out_ref, acc_ref, *, nsteps):
    i = pl.program_id(0)

    @pl.when(i == 0)
    def _init():
        acc_ref[...] = jnp.zeros_like(acc_ref)

    bounds = _recip_bounds()
    nblocks = NSTREAM * nsteps
    row = lax.broadcasted_iota(jnp.int32, (nblocks, NSTREAM), 0)
    col = lax.broadcasted_iota(jnp.int32, (nblocks, NSTREAM), 1)
    sel = (row == NSTREAM * i + col).astype(jnp.float32)
    lab4 = lax.dot_general(labT_ref[...], sel, (((1,), (0,)), ((), ())),
                           preferred_element_type=jnp.float32)

    for k, ref in enumerate((l0, l1, l2, l3)):
        x = ref[...]
        r = x.shape[0]
        m = jnp.max(x, axis=1, keepdims=True)
        e = jnp.exp(x - m)
        s = jnp.sum(e, axis=1, keepdims=True)
        conf = 1.0 / s
        pred = jnp.argmax(x, axis=1).reshape(r, 1)
        lab = lab4[:, k:k + 1]
        acc = (pred.astype(jnp.float32) == lab).astype(jnp.float32)

        gt = (s < bounds).astype(jnp.float32)
        acc_ref[0:1, :] += jnp.sum(gt, axis=0, keepdims=True)
        acc_ref[1:2, :] += jnp.sum(gt * acc, axis=0, keepdims=True)
        acc_ref[2:3, :] += jnp.sum(gt * conf, axis=0, keepdims=True)

    @pl.when(i == nsteps - 1)
    def _finish():
        out_ref[...] = acc_ref[...]


def _tc_partial(logits, labels):
    nblocks = NTC // BLOCK_ROWS
    nsteps = nblocks // NSTREAM
    c = logits.shape[1]
    labT = labels[:NTC].astype(jnp.float32).reshape(nblocks, BLOCK_ROWS).T

    return pl.pallas_call(
        functools.partial(_tc_kernel, nsteps=nsteps),
        grid=(nsteps,),
        in_specs=[
            pl.BlockSpec((BLOCK_ROWS, c), lambda i: (NSTREAM * i, 0)),
            pl.BlockSpec((BLOCK_ROWS, c), lambda i: (NSTREAM * i + 1, 0)),
            pl.BlockSpec((BLOCK_ROWS, c), lambda i: (NSTREAM * i + 2, 0)),
            pl.BlockSpec((BLOCK_ROWS, c), lambda i: (NSTREAM * i + 3, 0)),
            pl.BlockSpec((BLOCK_ROWS, nblocks), lambda i: (0, 0)),
        ],
        out_specs=pl.BlockSpec((8, 128), lambda i: (0, 0)),
        out_shape=jax.ShapeDtypeStruct((8, 128), jnp.float32),
        scratch_shapes=[pltpu.VMEM((8, 128), jnp.float32)],
    )(logits, logits, logits, logits, labT)


def _sc_body(logits_hbm, labels_hbm, out_hbm,
             xbuf0, xbuf1, lbuf0, lbuf1, bins, sem0, sem1):
    cc = lax.axis_index("c")
    ss = lax.axis_index("s")
    wid = ss * 2 + cc
    base = NTC + wid * SC_RPW

    zero16 = jnp.zeros((16,), jnp.float32)
    for j in range(3):
        bins[pl.ds(j * 16, 16)] = zero16

    lane = lax.broadcasted_iota(jnp.int32, (16,), 0)
    ones16 = jnp.ones((16,), jnp.float32)
    neg_inf = jnp.full((16,), -3.0e38, jnp.float32)

    def start_copy(t, xb, lb, sem):
        r0 = base + t * SC_CH
        pltpu.make_async_copy(logits_hbm.at[pl.ds(r0, SC_CH)], xb, sem).start()
        pltpu.make_async_copy(labels_hbm.at[pl.ds(r0, SC_CH)], lb, sem).start()

    def wait_copy(xb, lb, sem):
        pltpu.make_async_copy(logits_hbm.at[pl.ds(base, SC_CH)], xb, sem).wait()
        pltpu.make_async_copy(labels_hbm.at[pl.ds(base, SC_CH)], lb, sem).wait()

    def process(xb, lb):
        def group(g, __):
            rows16 = g * 16 + lane

            mv = [neg_inf] * 4
            pr = [jnp.full((16,), 0, jnp.int32)] * 4
            sa = [zero16] * 4
            for cidx in range(100):
                k = cidx % 4
                v = plsc.load_gather(xb, [rows16, jnp.full((16,), cidx, jnp.int32)])
                bigger = v > mv[k]
                pr[k] = jnp.where(bigger, jnp.full((16,), cidx, jnp.int32), pr[k])
                mv[k] = jnp.maximum(mv[k], v)
                sa[k] = sa[k] + jnp.exp(v)

            def merge(m0, p0, m1, p1):
                take1 = (m1 > m0) | ((m1 == m0) & (p1 < p0))
                return jnp.where(take1, m1, m0), jnp.where(take1, p1, p0)

            ma, pa = merge(mv[0], pr[0], mv[1], pr[1])
            mb, pb = merge(mv[2], pr[2], mv[3], pr[3])
            mval, prd = merge(ma, pa, mb, pb)
            sacc = (sa[0] + sa[1]) + (sa[2] + sa[3])

            conf = jnp.exp(mval) / sacc
            lab = lb[pl.ds(g * 16, 16)]
            accv = jnp.where(prd == lab, 1.0, 0.0).astype(jnp.float32)
            t15 = conf * 15.0
            ti = t15.astype(jnp.int32)
            tf = ti.astype(jnp.float32)
            b = jnp.where(tf == t15, ti - 1, ti)
            plsc.addupdate_scatter(bins, [b], ones16)
            plsc.addupdate_scatter(bins, [b + 16], accv)
            plsc.addupdate_scatter(bins, [b + 32], conf)
            return __

        lax.fori_loop(0, SC_CH // 16, group, 0)

    nchunks = SC_RPW // SC_CH
    start_copy(0, xbuf0, lbuf0, sem0)

    def pipelined(t, _):
        start_copy(2 * t + 1, xbuf1, lbuf1, sem1)
        wait_copy(xbuf0, lbuf0, sem0)
        process(xbuf0, lbuf0)

        @pl.when(t < nchunks // 2 - 1)
        def _refill():
            start_copy(2 * t + 2, xbuf0, lbuf0, sem0)

        wait_copy(xbuf1, lbuf1, sem1)
        process(xbuf1, lbuf1)
        return _

    lax.fori_loop(0, nchunks // 2, pipelined, 0)
    pltpu.sync_copy(bins, out_hbm.at[wid])


def _sc_partial(logits, labels):
    mesh = plsc.VectorSubcoreMesh(core_axis_name="c", subcore_axis_name="s")
    return pl.kernel(
        _sc_body,
        mesh=mesh,
        compiler_params=pltpu.CompilerParams(needs_layout_passes=False),
        out_type=jax.ShapeDtypeStruct((SC_WORKERS, 48), jnp.float32),
        scratch_types=[
            pltpu.VMEM((SC_CH, 100), jnp.float32),
            pltpu.VMEM((SC_CH, 100), jnp.float32),
            pltpu.VMEM((SC_CH,), jnp.int32),
            pltpu.VMEM((SC_CH,), jnp.int32),
            pltpu.VMEM((48,), jnp.float32),
            pltpu.SemaphoreType.DMA,
            pltpu.SemaphoreType.DMA,
        ],
    )(logits, labels)


def _combine_kernel(tc_ref, sc_ref, out_ref):
    cum = tc_ref[...]
    cnt = cum[0:1, 0:N_BINS] - cum[0:1, 1:N_BINS + 1]
    asum = cum[1:2, 0:N_BINS] - cum[1:2, 1:N_BINS + 1]
    csum = cum[2:3, 0:N_BINS] - cum[2:3, 1:N_BINS + 1]

    scs = jnp.sum(sc_ref[...], axis=0, keepdims=True)
    cnt = cnt + scs[0:1, 0:N_BINS]
    asum = asum + scs[0:1, 16:16 + N_BINS]
    csum = csum + scs[0:1, 32:32 + N_BINS]

    prop = cnt / jnp.float32(N_TOTAL)
    safe = jnp.maximum(cnt, 1.0)
    per_bin = jnp.where(cnt > 0.0, jnp.abs(csum / safe - asum / safe) * prop, 0.0)
    out_ref[...] = jnp.sum(per_bin).reshape(1, 1)


def _combine(tc_cum, sc_part):
    return pl.pallas_call(
        _combine_kernel,
        out_shape=jax.ShapeDtypeStruct((1, 1), jnp.float32),
    )(tc_cum, sc_part)


def kernel(logits, labels):
    sc_part = _sc_partial(logits, labels)
    tc_cum = _tc_partial(logits, labels)
    return _combine(tc_cum, sc_part).reshape(1)


# speedup vs baseline: 1.5560x; 1.0858x over previous
import functools
import jax
import jax.numpy as jnp
from jax import lax
from jax.experimental import pallas as pl
from jax.experimental.pallas import tpu as pltpu
from jax.experimental.pallas import tpu_sc as plsc

N_BINS = 15
NSTREAM = 4
BLOCK_ROWS = 2000

N_TOTAL = 1000000
SC_ROWS = 128000
NTC = N_TOTAL - SC_ROWS
SC_WORKERS = 32
SC_RPW = SC_ROWS // SC_WORKERS
SC_CH = 400


def _recip_bounds():
    lane = lax.broadcasted_iota(jnp.int32, (1, 128), 1)
    lane_f = lane.astype(jnp.float32)
    b = jnp.where(lane == 0, jnp.float32(3.0e38), 15.0 / lane_f)
    return jnp.where(lane <= N_BINS, b, jnp.float32(-1.0))


def _tc_kernel(l0, l1, l2, l3, labT_ref, out_ref, acc_ref, *, nsteps):
    i = pl.program_id(0)

    @pl.when(i == 0)
    def _init():
        acc_ref[...] = jnp.zeros_like(acc_ref)

    bounds = _recip_bounds()
    nblocks = NSTREAM * nsteps
    row = lax.broadcasted_iota(jnp.int32, (nblocks, NSTREAM), 0)
    col = lax.broadcasted_iota(jnp.int32, (nblocks, NSTREAM), 1)
    sel = (row == NSTREAM * i + col).astype(jnp.float32)
    lab4 = lax.dot_general(labT_ref[...], sel, (((1,), (0,)), ((), ())),
                           preferred_element_type=jnp.float32)

    for k, ref in enumerate((l0, l1, l2, l3)):
        x = ref[...]
        r = x.shape[0]
        m = jnp.max(x, axis=1, keepdims=True)
        e = jnp.exp(x - m)
        s = jnp.sum(e, axis=1, keepdims=True)
        conf = 1.0 / s
        pred = jnp.argmax(x, axis=1).reshape(r, 1)
        lab = lab4[:, k:k + 1]
        acc = (pred.astype(jnp.float32) == lab).astype(jnp.float32)

        gt = (s < bounds).astype(jnp.float32)
        acc_ref[0:1, :] += jnp.sum(gt, axis=0, keepdims=True)
        acc_ref[1:2, :] += jnp.sum(gt * acc, axis=0, keepdims=True)
        acc_ref[2:3, :] += jnp.sum(gt * conf, axis=0, keepdims=True)

    @pl.when(i == nsteps - 1)
    def _finish():
        out_ref[...] = acc_ref[...]


def _tc_partial(logits, labels):
    nblocks = NTC // BLOCK_ROWS
    nsteps = nblocks // NSTREAM
    c = logits.shape[1]
    labT = labels[:NTC].astype(jnp.float32).reshape(nblocks, BLOCK_ROWS).T

    return pl.pallas_call(
        functools.partial(_tc_kernel, nsteps=nsteps),
        grid=(nsteps,),
        in_specs=[
            pl.BlockSpec((BLOCK_ROWS, c), lambda i: (NSTREAM * i, 0)),
            pl.BlockSpec((BLOCK_ROWS, c), lambda i: (NSTREAM * i + 1, 0)),
            pl.BlockSpec((BLOCK_ROWS, c), lambda i: (NSTREAM * i + 2, 0)),
            pl.BlockSpec((BLOCK_ROWS, c), lambda i: (NSTREAM * i + 3, 0)),
            pl.BlockSpec((BLOCK_ROWS, nblocks), lambda i: (0, 0)),
        ],
        out_specs=pl.BlockSpec((8, 128), lambda i: (0, 0)),
        out_shape=jax.ShapeDtypeStruct((8, 128), jnp.float32),
        scratch_shapes=[pltpu.VMEM((8, 128), jnp.float32)],
    )(logits, logits, logits, logits, labT)


def _sc_body(logits_hbm, labels_hbm, out_hbm,
             xbuf0, xbuf1, lbuf0, lbuf1, bins, sem0, sem1):
    cc = lax.axis_index("c")
    ss = lax.axis_index("s")
    wid = ss * 2 + cc
    base = NTC + wid * SC_RPW

    zero16 = jnp.zeros((16,), jnp.float32)
    for j in range(3):
        bins[pl.ds(j * 16, 16)] = zero16

    lane = lax.broadcasted_iota(jnp.int32, (16,), 0)
    ones16 = jnp.ones((16,), jnp.float32)
    neg_inf = jnp.full((16,), -3.0e38, jnp.float32)

    def start_copy(t, xb, lb, sem):
        r0 = base + t * SC_CH
        pltpu.make_async_copy(logits_hbm.at[pl.ds(r0, SC_CH)], xb, sem).start()
        pltpu.make_async_copy(labels_hbm.at[pl.ds(r0, SC_CH)], lb, sem).start()

    def wait_copy(xb, lb, sem):
        pltpu.make_async_copy(logits_hbm.at[pl.ds(base, SC_CH)], xb, sem).wait()
        pltpu.make_async_copy(labels_hbm.at[pl.ds(base, SC_CH)], lb, sem).wait()

    def process(xb, lb):
        def group(g, __):
            rows16 = g * 16 + lane

            mv = [neg_inf] * 4
            pr = [jnp.full((16,), 0, jnp.int32)] * 4
            sa = [zero16] * 4
            for cidx in range(100):
                k = cidx % 4
                v = plsc.load_gather(xb, [rows16, jnp.full((16,), cidx, jnp.int32)])
                bigger = v > mv[k]
                pr[k] = jnp.where(bigger, jnp.full((16,), cidx, jnp.int32), pr[k])
                mv[k] = jnp.maximum(mv[k], v)
                sa[k] = sa[k] + jnp.exp(v)

            def merge(m0, p0, m1, p1):
                take1 = (m1 > m0) | ((m1 == m0) & (p1 < p0))
                return jnp.where(take1, m1, m0), jnp.where(take1, p1, p0)

            ma, pa = merge(mv[0], pr[0], mv[1], pr[1])
            mb, pb = merge(mv[2], pr[2], mv[3], pr[3])
            mval, prd = merge(ma, pa, mb, pb)
            sacc = (sa[0] + sa[1]) + (sa[2] + sa[3])

            conf = jnp.exp(mval) / sacc
            lab = lb[pl.ds(g * 16, 16)]
            accv = jnp.where(prd == lab, 1.0, 0.0).astype(jnp.float32)
            t15 = conf * 15.0
            ti = t15.astype(jnp.int32)
            tf = ti.astype(jnp.float32)
            b = jnp.where(tf == t15, ti - 1, ti)
            plsc.addupdate_scatter(bins, [b], ones16)
            plsc.addupdate_scatter(bins, [b + 16], accv)
            plsc.addupdate_scatter(bins, [b + 32], conf)
            return __

        lax.fori_loop(0, SC_CH // 16, group, 0)

    nchunks = SC_RPW // SC_CH
    start_copy(0, xbuf0, lbuf0, sem0)

    def pipelined(t, _):
        start_copy(2 * t + 1, xbuf1, lbuf1, sem1)
        wait_copy(xbuf0, lbuf0, sem0)
        process(xbuf0, lbuf0)

        @pl.when(t < nchunks // 2 - 1)
        def _refill():
            start_copy(2 * t + 2, xbuf0, lbuf0, sem0)

        wait_copy(xbuf1, lbuf1, sem1)
        process(xbuf1, lbuf1)
        return _

    lax.fori_loop(0, nchunks // 2, pipelined, 0)
    pltpu.sync_copy(bins, out_hbm.at[wid])


def _sc_partial(logits, labels):
    mesh = plsc.VectorSubcoreMesh(core_axis_name="c", subcore_axis_name="s")
    return pl.kernel(
        _sc_body,
        mesh=mesh,
        compiler_params=pltpu.CompilerParams(needs_layout_passes=False),
        out_type=jax.ShapeDtypeStruct((SC_WORKERS, 48), jnp.float32),
        scratch_types=[
            pltpu.VMEM((SC_CH, 100), jnp.float32),
            pltpu.VMEM((SC_CH, 100), jnp.float32),
            pltpu.VMEM((SC_CH,), jnp.int32),
            pltpu.VMEM((SC_CH,), jnp.int32),
            pltpu.VMEM((48,), jnp.float32),
            pltpu.SemaphoreType.DMA,
            pltpu.SemaphoreType.DMA,
        ],
    )(logits, labels)


def _combine_kernel(tc_ref, sc_ref, out_ref):
    cum = tc_ref[...]
    cnt = cum[0:1, 0:N_BINS] - cum[0:1, 1:N_BINS + 1]
    asum = cum[1:2, 0:N_BINS] - cum[1:2, 1:N_BINS + 1]
    csum = cum[2:3, 0:N_BINS] - cum[2:3, 1:N_BINS + 1]

    scs = jnp.sum(sc_ref[...], axis=0, keepdims=True)
    cnt = cnt + scs[0:1, 0:N_BINS]
    asum = asum + scs[0:1, 16:16 + N_BINS]
    csum = csum + scs[0:1, 32:32 + N_BINS]

    prop = cnt / jnp.float32(N_TOTAL)
    safe = jnp.maximum(cnt, 1.0)
    per_bin = jnp.where(cnt > 0.0, jnp.abs(csum / safe - asum / safe) * prop, 0.0)
    out_ref[...] = jnp.sum(per_bin).reshape(1, 1)


def _combine(tc_cum, sc_part):
    return pl.pallas_call(
        _combine_kernel,
        out_shape=jax.ShapeDtypeStruct((1, 1), jnp.float32),
    )(tc_cum, sc_part)


def kernel(logits, labels):
    sc_part = _sc_partial(logits, labels)
    tc_cum = _tc_partial(logits, labels)
    return _combine(tc_cum, sc_part).reshape(1)
